# baseline (speedup 1.0000x reference)
import functools

import jax
import jax.numpy as jnp
from jax import lax
from jax.experimental import pallas as pl
from jax.experimental.pallas import tpu as pltpu
from jax.experimental.pallas import tpu_sc as plsc

N_NODES = 10000
N_EDGES = 320000
D_FEAT = 128

NUM_CORES = 2
NUM_SUBCORES = 16
NUM_WORKERS = NUM_CORES * NUM_SUBCORES
EDGES_PER_WORKER = N_EDGES // NUM_WORKERS
CHUNK = 80
NUM_CHUNKS = EDGES_PER_WORKER // CHUNK
NBUF = 4
NUM_MAIN = (NUM_CHUNKS - 1) // NBUF * NBUF
assert NUM_CHUNKS - NUM_MAIN == 1

ROWS_PER_TILE = 624
TAIL_START = ROWS_PER_TILE * NUM_SUBCORES
TAIL_ROWS = N_NODES - TAIL_START
ZERO_ROWS = 16


def _sc_partial_sums(x, esd):
    mesh = plsc.VectorSubcoreMesh(core_axis_name="c", subcore_axis_name="s")

    scratch = (
        [pltpu.VMEM((2 * CHUNK,), jnp.int32) for _ in range(NBUF)]
        + [pltpu.VMEM((CHUNK, D_FEAT), jnp.float32) for _ in range(NBUF)]
        + [pltpu.VMEM_SHARED((N_NODES, D_FEAT), jnp.float32)]
        + [pltpu.SemaphoreType.DMA] * (3 * NBUF + 1)
    )

    @functools.partial(
        pl.kernel,
        mesh=mesh,
        out_type=jax.ShapeDtypeStruct((NUM_CORES, N_NODES, D_FEAT), jnp.float32),
        scratch_types=scratch,
    )
    def k(x_hbm, esd_hbm, out_hbm, *refs):
        idxb = refs[0:NBUF]
        rowsb = refs[NBUF : 2 * NBUF]
        acc_sh = refs[2 * NBUF]
        sem_i = refs[2 * NBUF + 1 : 2 * NBUF + 1 + NBUF]
        sem_g = refs[2 * NBUF + 1 + NBUF : 2 * NBUF + 1 + 2 * NBUF]
        sem_s = refs[2 * NBUF + 1 + 2 * NBUF : 2 * NBUF + 1 + 3 * NBUF]
        sem_z = refs[2 * NBUF + 1 + 3 * NBUF]

        cid = lax.axis_index("c")
        sid = lax.axis_index("s")
        wid = cid * NUM_SUBCORES + sid
        cbase = wid * NUM_CHUNKS

        zvec = jnp.zeros((16,), jnp.float32)
        zsrc = rowsb[NBUF - 1]
        for i in range(ZERO_ROWS):
            for j in range(D_FEAT // 16):
                zsrc[i, pl.ds(j * 16, 16)] = zvec
        row0 = sid * ROWS_PER_TILE
        nz = ROWS_PER_TILE // ZERO_ROWS

        def zdst(i):
            return acc_sh.at[pl.ds(row0 + i * ZERO_ROWS, ZERO_ROWS)]

        zsl = zsrc.at[pl.ds(0, ZERO_ROWS)]
        for i in range(nz):
            pltpu.async_copy(zsl, zdst(i), sem_z)

        @pl.when(sid == 0)
        def _zero_tail():
            pltpu.async_copy(zsl, acc_sh.at[pl.ds(TAIL_START, TAIL_ROWS)], sem_z)

        for i in range(nz):
            pltpu.make_async_copy(zsl, zdst(i), sem_z).wait()

        @pl.when(sid == 0)
        def _zero_tail_wait():
            pltpu.make_async_copy(
                zsl, acc_sh.at[pl.ds(TAIL_START, TAIL_ROWS)], sem_z
            ).wait()

        plsc.subcore_barrier()

        def issue_idx(c, sl):
            pltpu.async_copy(
                esd_hbm.at[pl.ds((cbase + c) * 2 * CHUNK, 2 * CHUNK)],
                idxb[sl],
                sem_i[sl],
            )

        def wait_idx(c, sl):
            pltpu.make_async_copy(
                esd_hbm.at[pl.ds((cbase + c) * 2 * CHUNK, 2 * CHUNK)],
                idxb[sl],
                sem_i[sl],
            ).wait()

        def issue_gather(sl):
            pltpu.async_copy(
                x_hbm.at[idxb[sl].at[pl.ds(0, CHUNK)]], rowsb[sl], sem_g[sl]
            )

        def wait_gather(sl):
            pltpu.make_async_copy(
                x_hbm.at[idxb[sl].at[pl.ds(0, CHUNK)]], rowsb[sl], sem_g[sl]
            ).wait()

        def issue_scatter(sl):
            pltpu.async_copy(
                rowsb[sl],
                acc_sh.at[idxb[sl].at[pl.ds(CHUNK, CHUNK)]],
                sem_s[sl],
                add=True,
            )

        def wait_scatter(sl):
            pltpu.make_async_copy(
                rowsb[sl],
                acc_sh.at[idxb[sl].at[pl.ds(CHUNK, CHUNK)]],
                sem_s[sl],
            ).wait()

        def body(g, carry):
            for b in range(NBUF):
                i = g + b
                sl = b
                sl1 = (b - 1) % NBUF
                sl2 = (b - 2) % NBUF

                @pl.when(i >= NBUF)
                def _drain():
                    wait_scatter(sl)

                issue_idx(i, sl)

                @pl.when(i >= 1)
                def _gather():
                    wait_idx(i - 1, sl1)
                    issue_gather(sl1)

                @pl.when(i >= 2)
                def _scatter():
                    wait_gather(sl2)
                    issue_scatter(sl2)

            return carry

        lax.fori_loop(0, NUM_MAIN // NBUF, lambda g, c: body(g * NBUF, c), 0)

        last = NUM_CHUNKS - 1
        wait_scatter(0)
        issue_idx(last, 0)
        wait_idx(last - 1, 3)
        issue_gather(3)
        wait_gather(2)
        issue_scatter(2)
        wait_idx(last, 0)
        issue_gather(0)
        wait_gather(3)
        issue_scatter(3)
        wait_gather(0)
        issue_scatter(0)
        for b in (1, 2, 3, 0):
            wait_scatter(b)

        plsc.subcore_barrier()

        pltpu.sync_copy(
            acc_sh.at[pl.ds(row0, ROWS_PER_TILE)],
            out_hbm.at[cid, pl.ds(row0, ROWS_PER_TILE)],
        )

        @pl.when(sid == 0)
        def _write_tail():
            pltpu.sync_copy(
                acc_sh.at[pl.ds(TAIL_START, TAIL_ROWS)],
                out_hbm.at[cid, pl.ds(TAIL_START, TAIL_ROWS)],
            )

    return k(x, esd)


def _tc_add(partials):
    grid = 10
    rows = N_NODES // grid

    def add_kernel(a_ref, o_ref):
        o_ref[...] = a_ref[0] + a_ref[1]

    return pl.pallas_call(
        add_kernel,
        out_shape=jax.ShapeDtypeStruct((N_NODES, D_FEAT), jnp.float32),
        grid=(grid,),
        in_specs=[
            pl.BlockSpec((NUM_CORES, rows, D_FEAT), lambda i: (0, i, 0))
        ],
        out_specs=pl.BlockSpec((rows, D_FEAT), lambda i: (i, 0)),
    )(partials)


def kernel(x, edge_index):
    nchunks = N_EDGES // CHUNK
    esd = jnp.stack(
        [edge_index[0].reshape(nchunks, CHUNK),
         edge_index[1].reshape(nchunks, CHUNK)],
        axis=1,
    ).reshape(-1)
    partials = _sc_partial_sums(x, esd)
    return _tc_add(partials)


# speedup vs baseline: 1.2296x; 1.2296x over previous
import functools

import jax
import jax.numpy as jnp
from jax import lax
from jax.experimental import pallas as pl
from jax.experimental.pallas import tpu as pltpu
from jax.experimental.pallas import tpu_sc as plsc

N_NODES = 10000
N_EDGES = 320000
D_FEAT = 128

NUM_CORES = 2
NUM_SUBCORES = 16
NUM_WORKERS = NUM_CORES * NUM_SUBCORES
CHUNK = 128
NUM_GLOBAL_CHUNKS = N_EDGES // CHUNK
NUM_CHUNKS = NUM_GLOBAL_CHUNKS // NUM_WORKERS
NUM_EXTRA = NUM_GLOBAL_CHUNKS - NUM_CHUNKS * NUM_WORKERS
NBUF = 3
assert NUM_CHUNKS % NBUF == 0

ROWS_PER_TILE = 624
TAIL_START = ROWS_PER_TILE * NUM_SUBCORES
TAIL_ROWS = N_NODES - TAIL_START
ZERO_ROWS = 16


def _sc_partial_sums(x, edge_index):
    mesh = plsc.VectorSubcoreMesh(core_axis_name="c", subcore_axis_name="s")

    scratch = (
        [pltpu.VMEM((2, CHUNK), jnp.int32) for _ in range(NBUF)]
        + [pltpu.VMEM((CHUNK, D_FEAT), jnp.float32) for _ in range(NBUF)]
        + [pltpu.VMEM_SHARED((N_NODES, D_FEAT), jnp.float32)]
        + [pltpu.SemaphoreType.DMA] * (3 * NBUF + 1)
    )

    @functools.partial(
        pl.kernel,
        mesh=mesh,
        out_type=jax.ShapeDtypeStruct((NUM_CORES, N_NODES, D_FEAT), jnp.float32),
        scratch_types=scratch,
    )
    def k(x_hbm, ei_hbm, out_hbm, *refs):
        idxb = refs[0:NBUF]
        rowsb = refs[NBUF : 2 * NBUF]
        acc_sh = refs[2 * NBUF]
        sem_i = refs[2 * NBUF + 1 : 2 * NBUF + 1 + NBUF]
        sem_g = refs[2 * NBUF + 1 + NBUF : 2 * NBUF + 1 + 2 * NBUF]
        sem_s = refs[2 * NBUF + 1 + 2 * NBUF : 2 * NBUF + 1 + 3 * NBUF]
        sem_z = refs[2 * NBUF + 1 + 3 * NBUF]

        cid = lax.axis_index("c")
        sid = lax.axis_index("s")
        wid = cid * NUM_SUBCORES + sid

        zvec = jnp.zeros((16,), jnp.float32)
        zsrc = rowsb[NBUF - 1]
        for i in range(ZERO_ROWS):
            for j in range(D_FEAT // 16):
                zsrc[i, pl.ds(j * 16, 16)] = zvec
        row0 = sid * ROWS_PER_TILE
        nz = ROWS_PER_TILE // ZERO_ROWS

        def zdst(i):
            return acc_sh.at[pl.ds(row0 + i * ZERO_ROWS, ZERO_ROWS)]

        zsl = zsrc.at[pl.ds(0, ZERO_ROWS)]
        for i in range(nz):
            pltpu.async_copy(zsl, zdst(i), sem_z)

        @pl.when(sid == 0)
        def _zero_tail():
            pltpu.async_copy(zsl, acc_sh.at[pl.ds(TAIL_START, TAIL_ROWS)], sem_z)

        for i in range(nz):
            pltpu.make_async_copy(zsl, zdst(i), sem_z).wait()

        @pl.when(sid == 0)
        def _zero_tail_wait():
            pltpu.make_async_copy(
                zsl, acc_sh.at[pl.ds(TAIL_START, TAIL_ROWS)], sem_z
            ).wait()

        plsc.subcore_barrier()

        def idx_src(c):
            return ei_hbm.at[:, pl.ds((wid + NUM_WORKERS * c) * CHUNK, CHUNK)]

        def issue_idx(c, sl):
            pltpu.async_copy(idx_src(c), idxb[sl], sem_i[sl])

        def wait_idx(c, sl):
            pltpu.make_async_copy(idx_src(c), idxb[sl], sem_i[sl]).wait()

        def issue_gather(sl):
            pltpu.async_copy(x_hbm.at[idxb[sl].at[0]], rowsb[sl], sem_g[sl])

        def wait_gather(sl):
            pltpu.make_async_copy(
                x_hbm.at[idxb[sl].at[0]], rowsb[sl], sem_g[sl]
            ).wait()

        def issue_scatter(sl):
            pltpu.async_copy(
                rowsb[sl], acc_sh.at[idxb[sl].at[1]], sem_s[sl], add=True
            )

        def wait_scatter(sl):
            pltpu.make_async_copy(
                rowsb[sl], acc_sh.at[idxb[sl].at[1]], sem_s[sl]
            ).wait()

        def body(g, carry):
            for b in range(NBUF):
                i = g + b
                sl = b
                sl1 = (b - 1) % NBUF
                sl2 = (b - 2) % NBUF

                @pl.when(i >= NBUF)
                def _drain():
                    wait_scatter(sl)

                issue_idx(i, sl)

                @pl.when(i >= 1)
                def _gather():
                    wait_idx(i - 1, sl1)
                    issue_gather(sl1)

                @pl.when(i >= 2)
                def _scatter():
                    wait_gather(sl2)
                    issue_scatter(sl2)

            return carry

        lax.fori_loop(0, NUM_CHUNKS // NBUF, lambda g, c: body(g * NBUF, c), 0)

        last = NUM_CHUNKS - 1
        sl_last = last % NBUF
        sl_prev = (last - 1) % NBUF
        wait_idx(last, sl_last)
        issue_gather(sl_last)
        wait_gather(sl_prev)
        issue_scatter(sl_prev)
        wait_gather(sl_last)
        issue_scatter(sl_last)
        for j in range(NBUF):
            wait_scatter((last - j) % NBUF)

        @pl.when(wid < NUM_EXTRA)
        def _extra():
            xsrc = ei_hbm.at[
                :, pl.ds((NUM_CHUNKS * NUM_WORKERS + wid) * CHUNK, CHUNK)
            ]
            pltpu.sync_copy(xsrc, idxb[0])
            pltpu.async_copy(x_hbm.at[idxb[0].at[0]], rowsb[0], sem_g[0]).wait()
            pltpu.sync_copy(rowsb[0], acc_sh.at[idxb[0].at[1]], add=True)

        plsc.subcore_barrier()

        pltpu.sync_copy(
            acc_sh.at[pl.ds(row0, ROWS_PER_TILE)],
            out_hbm.at[cid, pl.ds(row0, ROWS_PER_TILE)],
        )

        @pl.when(sid == 0)
        def _write_tail():
            pltpu.sync_copy(
                acc_sh.at[pl.ds(TAIL_START, TAIL_ROWS)],
                out_hbm.at[cid, pl.ds(TAIL_START, TAIL_ROWS)],
            )

    return k(x, edge_index)


def _tc_add(partials):
    grid = 10
    rows = N_NODES // grid

    def add_kernel(a_ref, o_ref):
        o_ref[...] = a_ref[0] + a_ref[1]

    return pl.pallas_call(
        add_kernel,
        out_shape=jax.ShapeDtypeStruct((N_NODES, D_FEAT), jnp.float32),
        grid=(grid,),
        in_specs=[
            pl.BlockSpec((NUM_CORES, rows, D_FEAT), lambda i: (0, i, 0))
        ],
        out_specs=pl.BlockSpec((rows, D_FEAT), lambda i: (i, 0)),
    )(partials)


def kernel(x, edge_index):
    partials = _sc_partial_sums(x, edge_index)
    return _tc_add(partials)


# speedup vs baseline: 1.2434x; 1.0112x over previous
import functools

import jax
import jax.numpy as jnp
from jax import lax
from jax.experimental import pallas as pl
from jax.experimental.pallas import tpu as pltpu
from jax.experimental.pallas import tpu_sc as plsc

N_NODES = 10000
N_EDGES = 320000
D_FEAT = 128

NUM_CORES = 2
NUM_SUBCORES = 16
NUM_WORKERS = NUM_CORES * NUM_SUBCORES
CHUNK = 128
NUM_GLOBAL_CHUNKS = N_EDGES // CHUNK
NUM_CHUNKS = NUM_GLOBAL_CHUNKS // NUM_WORKERS
NUM_EXTRA = NUM_GLOBAL_CHUNKS - NUM_CHUNKS * NUM_WORKERS
NBUF = 3
assert NUM_CHUNKS % NBUF == 0

ROWS_PER_TILE = 624
TAIL_START = ROWS_PER_TILE * NUM_SUBCORES
TAIL_ROWS = N_NODES - TAIL_START
ZERO_ROWS = 16


def _sc_partial_sums(x, edge_index):
    mesh = plsc.VectorSubcoreMesh(core_axis_name="c", subcore_axis_name="s")

    scratch = (
        [pltpu.VMEM((2, CHUNK), jnp.int32) for _ in range(NBUF)]
        + [pltpu.VMEM((CHUNK, D_FEAT), jnp.float32) for _ in range(NBUF)]
        + [pltpu.VMEM_SHARED((N_NODES, D_FEAT), jnp.float32)]
        + [pltpu.SemaphoreType.DMA] * (3 * NBUF + 1)
    )

    @functools.partial(
        pl.kernel,
        mesh=mesh,
        out_type=jax.ShapeDtypeStruct((NUM_CORES, N_NODES, D_FEAT), jnp.float32),
        scratch_types=scratch,
    )
    def k(x_hbm, ei_hbm, out_hbm, *refs):
        idxb = refs[0:NBUF]
        rowsb = refs[NBUF : 2 * NBUF]
        acc_sh = refs[2 * NBUF]
        sem_i = refs[2 * NBUF + 1 : 2 * NBUF + 1 + NBUF]
        sem_g = refs[2 * NBUF + 1 + NBUF : 2 * NBUF + 1 + 2 * NBUF]
        sem_s = refs[2 * NBUF + 1 + 2 * NBUF : 2 * NBUF + 1 + 3 * NBUF]
        sem_z = refs[2 * NBUF + 1 + 3 * NBUF]

        cid = lax.axis_index("c")
        sid = lax.axis_index("s")
        wid = cid * NUM_SUBCORES + sid

        zvec = jnp.zeros((16,), jnp.float32)
        zsrc = rowsb[NBUF - 1]
        for i in range(ZERO_ROWS):
            for j in range(D_FEAT // 16):
                zsrc[i, pl.ds(j * 16, 16)] = zvec
        row0 = sid * ROWS_PER_TILE
        nz = ROWS_PER_TILE // ZERO_ROWS

        def zdst(i):
            return acc_sh.at[pl.ds(row0 + i * ZERO_ROWS, ZERO_ROWS)]

        zsl = zsrc.at[pl.ds(0, ZERO_ROWS)]
        for i in range(nz):
            pltpu.async_copy(zsl, zdst(i), sem_z)

        @pl.when(sid == 0)
        def _zero_tail():
            pltpu.async_copy(zsl, acc_sh.at[pl.ds(TAIL_START, TAIL_ROWS)], sem_z)

        def idx_src(c):
            return ei_hbm.at[:, pl.ds((wid + NUM_WORKERS * c) * CHUNK, CHUNK)]

        def issue_idx(c, sl):
            pltpu.async_copy(idx_src(c), idxb[sl], sem_i[sl])

        def wait_idx(c, sl):
            pltpu.make_async_copy(idx_src(c), idxb[sl], sem_i[sl]).wait()

        def issue_gather(sl):
            pltpu.async_copy(x_hbm.at[idxb[sl].at[0]], rowsb[sl], sem_g[sl])

        def wait_gather(sl):
            pltpu.make_async_copy(
                x_hbm.at[idxb[sl].at[0]], rowsb[sl], sem_g[sl]
            ).wait()

        def issue_scatter(sl):
            pltpu.async_copy(
                rowsb[sl], acc_sh.at[idxb[sl].at[1]], sem_s[sl], add=True
            )

        def wait_scatter(sl):
            pltpu.make_async_copy(
                rowsb[sl], acc_sh.at[idxb[sl].at[1]], sem_s[sl]
            ).wait()

        def body(g, carry):
            for b in range(NBUF):
                i = g + b
                sl = b
                sl1 = (b - 1) % NBUF
                sl2 = (b - 2) % NBUF

                @pl.when(i >= NBUF)
                def _drain():
                    wait_scatter(sl)

                issue_idx(i, sl)

                @pl.when(i >= 1)
                def _gather():
                    wait_idx(i - 1, sl1)
                    issue_gather(sl1)

                @pl.when(i >= 2)
                def _scatter():
                    wait_gather(sl2)
                    issue_scatter(sl2)

            return carry

        issue_idx(0, 0)
        issue_idx(1, 1)
        issue_idx(2, 2)
        wait_idx(0, 0)
        issue_gather(0)
        wait_idx(1, 1)
        issue_gather(1)

        for i in range(nz):
            pltpu.make_async_copy(zsl, zdst(i), sem_z).wait()

        @pl.when(sid == 0)
        def _zero_tail_wait():
            pltpu.make_async_copy(
                zsl, acc_sh.at[pl.ds(TAIL_START, TAIL_ROWS)], sem_z
            ).wait()

        plsc.subcore_barrier()

        wait_gather(0)
        issue_scatter(0)

        lax.fori_loop(1, NUM_CHUNKS // NBUF, lambda g, c: body(g * NBUF, c), 0)

        last = NUM_CHUNKS - 1
        sl_last = last % NBUF
        sl_prev = (last - 1) % NBUF
        wait_idx(last, sl_last)
        issue_gather(sl_last)
        wait_gather(sl_prev)
        issue_scatter(sl_prev)
        wait_gather(sl_last)
        issue_scatter(sl_last)
        for j in range(NBUF):
            wait_scatter((last - j) % NBUF)

        @pl.when(wid < NUM_EXTRA)
        def _extra():
            xsrc = ei_hbm.at[
                :, pl.ds((NUM_CHUNKS * NUM_WORKERS + wid) * CHUNK, CHUNK)
            ]
            pltpu.sync_copy(xsrc, idxb[0])
            pltpu.async_copy(x_hbm.at[idxb[0].at[0]], rowsb[0], sem_g[0]).wait()
            pltpu.sync_copy(rowsb[0], acc_sh.at[idxb[0].at[1]], add=True)

        plsc.subcore_barrier()

        pltpu.sync_copy(
            acc_sh.at[pl.ds(row0, ROWS_PER_TILE)],
            out_hbm.at[cid, pl.ds(row0, ROWS_PER_TILE)],
        )

        @pl.when(sid == 0)
        def _write_tail():
            pltpu.sync_copy(
                acc_sh.at[pl.ds(TAIL_START, TAIL_ROWS)],
                out_hbm.at[cid, pl.ds(TAIL_START, TAIL_ROWS)],
            )

    return k(x, edge_index)


def _tc_add(partials):
    grid = 10
    rows = N_NODES // grid

    def add_kernel(a_ref, o_ref):
        o_ref[...] = a_ref[0] + a_ref[1]

    return pl.pallas_call(
        add_kernel,
        out_shape=jax.ShapeDtypeStruct((N_NODES, D_FEAT), jnp.float32),
        grid=(grid,),
        in_specs=[
            pl.BlockSpec((NUM_CORES, rows, D_FEAT), lambda i: (0, i, 0))
        ],
        out_specs=pl.BlockSpec((rows, D_FEAT), lambda i: (i, 0)),
    )(partials)


def kernel(x, edge_index):
    partials = _sc_partial_sums(x, edge_index)
    return _tc_add(partials)
